# baseline (speedup 1.0000x reference)
import functools

import jax
import jax.numpy as jnp
from jax.experimental import pallas as pl
from jax.experimental.pallas import tpu as pltpu

_ROWS = 400
_ROWS_Q = 1024
_SPLIT = 6912


def _support_kernel(x_ref, w_ref, out_ref):
    out_ref[...] = jnp.dot(x_ref[...], w_ref[...],
                           preferred_element_type=jnp.float32)


def _sweep1_kernel(adj_ref, s_ref, b_ref, out_ref, adjq8_ref, adjqb_ref,
                   cs_ref):
    a = adj_ref[...]
    adjq8_ref[...] = jnp.round(a[:, :_SPLIT] * 255.0 - 128.0).astype(jnp.int8)
    adjqb_ref[...] = a[:, _SPLIT:].astype(jnp.bfloat16)
    s_next = jnp.dot(a, s_ref[...],
                     preferred_element_type=jnp.float32) + b_ref[...]
    out_ref[...] = s_next.astype(jnp.bfloat16)
    row = (jax.lax.broadcasted_iota(jnp.int32, s_next.shape, 0)
           + pl.program_id(0) * _ROWS)
    part = jnp.sum(jnp.where(row < _SPLIT, s_next, 0.0),
                   axis=0, keepdims=True)

    @pl.when(pl.program_id(0) == 0)
    def _init():
        cs_ref[...] = part

    @pl.when(pl.program_id(0) != 0)
    def _acc():
        cs_ref[...] += part


def _sweeps_q_fused_kernel(n_rows, adjq8_ref, adjqb_ref, s1_ref, cs1_ref,
                           b_ref, out_ref, sa_ref, sb_ref, csa_ref, csb_ref):
    k = pl.program_id(0)
    i = pl.program_id(1)

    @pl.when((k == 0) & (i == 0))
    def _preload():
        sb_ref[0:n_rows, :] = s1_ref[...]
        csb_ref[...] = cs1_ref[...]

    even = (k % 2) == 0
    src = jnp.where(even, sb_ref[0:n_rows, :], sa_ref[0:n_rows, :])
    cs_in = jnp.where(even, csb_ref[...], csa_ref[...])
    raw8 = jnp.dot(adjq8_ref[...].astype(jnp.bfloat16), src[:_SPLIT, :],
                   preferred_element_type=jnp.float32)
    rawb = jnp.dot(adjqb_ref[...], src[_SPLIT:, :],
                   preferred_element_type=jnp.float32)
    s_next = ((raw8 + 128.0 * cs_in) * (1.0 / 255.0) + rawb + b_ref[0])

    row = (jax.lax.broadcasted_iota(jnp.int32, s_next.shape, 0)
           + i * _ROWS_Q)
    part = jnp.sum(jnp.where(row < _SPLIT, s_next, 0.0),
                   axis=0, keepdims=True)

    @pl.when(k == 0)
    def _store_a():
        sa_ref[pl.ds(i * _ROWS_Q, _ROWS_Q), :] = s_next.astype(jnp.bfloat16)

    @pl.when(k == 1)
    def _store_b():
        sb_ref[pl.ds(i * _ROWS_Q, _ROWS_Q), :] = s_next.astype(jnp.bfloat16)

    @pl.when((k == 0) & (i == 0))
    def _cs_a_init():
        csa_ref[...] = part

    @pl.when((k == 0) & (i != 0))
    def _cs_a_acc():
        csa_ref[...] += part

    @pl.when((k == 1) & (i == 0))
    def _cs_b_init():
        csb_ref[...] = part

    @pl.when((k == 1) & (i != 0))
    def _cs_b_acc():
        csb_ref[...] += part

    @pl.when(k < 2)
    def _out_passthrough():
        out_ref[...] = s_next

    @pl.when(k == 2)
    def _out_final():
        m = jnp.max(s_next, axis=1, keepdims=True)
        lse = jnp.log(jnp.sum(jnp.exp(s_next - m), axis=1, keepdims=True))
        out_ref[...] = s_next - m - lse


def _support(x, w):
    n, _ = x.shape
    h = w.shape[1]
    return pl.pallas_call(
        _support_kernel,
        out_shape=jax.ShapeDtypeStruct((n, h), jnp.float32),
    )(x, w)


def _sweep1(adj, s, beta):
    n = adj.shape[0]
    h = s.shape[1]
    grid = (n // _ROWS,)
    return pl.pallas_call(
        _sweep1_kernel,
        grid=grid,
        in_specs=[
            pl.BlockSpec((_ROWS, n), lambda i: (i, 0)),
            pl.BlockSpec((n, h), lambda i: (0, 0)),
            pl.BlockSpec((1, h), lambda i: (0, 0)),
        ],
        out_specs=[
            pl.BlockSpec((_ROWS, h), lambda i: (i, 0)),
            pl.BlockSpec((_ROWS, _SPLIT), lambda i: (i, 0)),
            pl.BlockSpec((_ROWS, n - _SPLIT), lambda i: (i, 0)),
            pl.BlockSpec((1, h), lambda i: (0, 0)),
        ],
        out_shape=[
            jax.ShapeDtypeStruct((n, h), jnp.bfloat16),
            jax.ShapeDtypeStruct((n, _SPLIT), jnp.int8),
            jax.ShapeDtypeStruct((n, n - _SPLIT), jnp.bfloat16),
            jax.ShapeDtypeStruct((1, h), jnp.float32),
        ],
    )(adj, s, beta)


def _sweeps_q_fused(adjq8, adjqb, s1, cs1, betas):
    n = adjq8.shape[0]
    h = s1.shape[1]
    n_blocks = pl.cdiv(n, _ROWS_Q)
    n_pad = n_blocks * _ROWS_Q
    return pl.pallas_call(
        functools.partial(_sweeps_q_fused_kernel, n),
        grid=(3, n_blocks),
        in_specs=[
            pl.BlockSpec((_ROWS_Q, _SPLIT), lambda k, i: (i, 0)),
            pl.BlockSpec((_ROWS_Q, n - _SPLIT), lambda k, i: (i, 0)),
            pl.BlockSpec((n, h), lambda k, i: (0, 0)),
            pl.BlockSpec((1, h), lambda k, i: (0, 0)),
            pl.BlockSpec((1, 1, h), lambda k, i: (k, 0, 0)),
        ],
        out_specs=pl.BlockSpec((_ROWS_Q, h), lambda k, i: (i, 0)),
        out_shape=jax.ShapeDtypeStruct((n, h), jnp.float32),
        scratch_shapes=[
            pltpu.VMEM((n_pad, h), jnp.bfloat16),
            pltpu.VMEM((n_pad, h), jnp.bfloat16),
            pltpu.VMEM((1, h), jnp.float32),
            pltpu.VMEM((1, h), jnp.float32),
        ],
    )(adjq8, adjqb, s1, cs1, betas)


def kernel(h, adj, W_in, b_in, W0, b0, W1, b1, W_out, b_out):
    Wc = W_in @ (W0 @ (W1 @ W_out))
    beta0 = (b_in @ (W0 @ (W1 @ W_out))).reshape(1, -1)
    beta1 = (b0 @ (W1 @ W_out)).reshape(1, -1)
    beta2 = (b1 @ W_out).reshape(1, -1)
    beta3 = b_out.reshape(1, -1)

    s = _support(h, Wc)
    s, adj_q8, adj_qb, cs = _sweep1(adj, s, beta0)
    betas = jnp.concatenate([beta1, beta2, beta3], axis=0).reshape(3, 1, -1)
    return _sweeps_q_fused(adj_q8, adj_qb, s, cs, betas)


# speedup vs baseline: 1.0374x; 1.0374x over previous
import functools

import jax
import jax.numpy as jnp
from jax.experimental import pallas as pl
from jax.experimental.pallas import tpu as pltpu

_ROWS = 480
_ROWS_Q = 1024


def _support_kernel(x_ref, w_ref, out_ref):
    out_ref[...] = jnp.dot(x_ref[...], w_ref[...],
                           preferred_element_type=jnp.float32)


def _sweep1_kernel(n_rows, adj_ref, s_ref, b_ref, out_ref, adjq_ref, cs_ref):
    a = adj_ref[...]
    adjq_ref[...] = jnp.round(a * 255.0 - 128.0).astype(jnp.int8)
    s_next = jnp.dot(a, s_ref[...],
                     preferred_element_type=jnp.float32) + b_ref[...]
    out_ref[...] = s_next.astype(jnp.bfloat16)
    row = (jax.lax.broadcasted_iota(jnp.int32, s_next.shape, 0)
           + pl.program_id(0) * _ROWS)
    part = jnp.sum(jnp.where(row < n_rows, s_next, 0.0),
                   axis=0, keepdims=True)

    @pl.when(pl.program_id(0) == 0)
    def _init():
        cs_ref[...] = part

    @pl.when(pl.program_id(0) != 0)
    def _acc():
        cs_ref[...] += part


def _sweeps_q_fused_kernel(n_rows, adjq_ref, s1_ref, cs1_ref, b_ref,
                           out_ref, sa_ref, sb_ref, csa_ref, csb_ref):
    k = pl.program_id(0)
    i = pl.program_id(1)

    @pl.when((k == 0) & (i == 0))
    def _preload():
        sb_ref[0:n_rows, :] = s1_ref[...]
        csb_ref[...] = cs1_ref[...]

    even = (k % 2) == 0
    src = jnp.where(even, sb_ref[0:n_rows, :], sa_ref[0:n_rows, :])
    cs_in = jnp.where(even, csb_ref[...], csa_ref[...])
    raw = jnp.dot(adjq_ref[...].astype(jnp.bfloat16), src,
                  preferred_element_type=jnp.float32)
    s_next = (raw + 128.0 * cs_in) * (1.0 / 255.0) + b_ref[0]

    row = (jax.lax.broadcasted_iota(jnp.int32, s_next.shape, 0)
           + i * _ROWS_Q)
    part = jnp.sum(jnp.where(row < n_rows, s_next, 0.0),
                   axis=0, keepdims=True)

    @pl.when(k == 0)
    def _store_a():
        sa_ref[pl.ds(i * _ROWS_Q, _ROWS_Q), :] = s_next.astype(jnp.bfloat16)

    @pl.when(k == 1)
    def _store_b():
        sb_ref[pl.ds(i * _ROWS_Q, _ROWS_Q), :] = s_next.astype(jnp.bfloat16)

    @pl.when((k == 0) & (i == 0))
    def _cs_a_init():
        csa_ref[...] = part

    @pl.when((k == 0) & (i != 0))
    def _cs_a_acc():
        csa_ref[...] += part

    @pl.when((k == 1) & (i == 0))
    def _cs_b_init():
        csb_ref[...] = part

    @pl.when((k == 1) & (i != 0))
    def _cs_b_acc():
        csb_ref[...] += part

    @pl.when(k < 2)
    def _out_passthrough():
        out_ref[...] = s_next

    @pl.when(k == 2)
    def _out_final():
        m = jnp.max(s_next, axis=1, keepdims=True)
        lse = jnp.log(jnp.sum(jnp.exp(s_next - m), axis=1, keepdims=True))
        out_ref[...] = s_next - m - lse


def _support(x, w):
    n, _ = x.shape
    h = w.shape[1]
    return pl.pallas_call(
        _support_kernel,
        out_shape=jax.ShapeDtypeStruct((n, h), jnp.float32),
    )(x, w)


def _sweep1(adj, s, beta):
    n = adj.shape[0]
    h = s.shape[1]
    grid = (pl.cdiv(n, _ROWS),)
    return pl.pallas_call(
        functools.partial(_sweep1_kernel, n),
        grid=grid,
        in_specs=[
            pl.BlockSpec((_ROWS, n), lambda i: (i, 0)),
            pl.BlockSpec((n, h), lambda i: (0, 0)),
            pl.BlockSpec((1, h), lambda i: (0, 0)),
        ],
        out_specs=[
            pl.BlockSpec((_ROWS, h), lambda i: (i, 0)),
            pl.BlockSpec((_ROWS, n), lambda i: (i, 0)),
            pl.BlockSpec((1, h), lambda i: (0, 0)),
        ],
        out_shape=[
            jax.ShapeDtypeStruct((n, h), jnp.bfloat16),
            jax.ShapeDtypeStruct((n, n), jnp.int8),
            jax.ShapeDtypeStruct((1, h), jnp.float32),
        ],
    )(adj, s, beta)


def _sweeps_q_fused(adjq, s1, cs1, betas):
    n = adjq.shape[0]
    h = s1.shape[1]
    n_blocks = pl.cdiv(n, _ROWS_Q)
    n_pad = n_blocks * _ROWS_Q
    return pl.pallas_call(
        functools.partial(_sweeps_q_fused_kernel, n),
        grid=(3, n_blocks),
        in_specs=[
            pl.BlockSpec((_ROWS_Q, n), lambda k, i: (i, 0)),
            pl.BlockSpec((n, h), lambda k, i: (0, 0)),
            pl.BlockSpec((1, h), lambda k, i: (0, 0)),
            pl.BlockSpec((1, 1, h), lambda k, i: (k, 0, 0)),
        ],
        out_specs=pl.BlockSpec((_ROWS_Q, h), lambda k, i: (i, 0)),
        out_shape=jax.ShapeDtypeStruct((n, h), jnp.float32),
        scratch_shapes=[
            pltpu.VMEM((n_pad, h), jnp.bfloat16),
            pltpu.VMEM((n_pad, h), jnp.bfloat16),
            pltpu.VMEM((1, h), jnp.float32),
            pltpu.VMEM((1, h), jnp.float32),
        ],
    )(adjq, s1, cs1, betas)


def kernel(h, adj, W_in, b_in, W0, b0, W1, b1, W_out, b_out):
    Wc = W_in @ (W0 @ (W1 @ W_out))
    beta0 = (b_in @ (W0 @ (W1 @ W_out))).reshape(1, -1)
    beta1 = (b0 @ (W1 @ W_out)).reshape(1, -1)
    beta2 = (b1 @ W_out).reshape(1, -1)
    beta3 = b_out.reshape(1, -1)

    s = _support(h, Wc)
    s, adj_q, cs = _sweep1(adj, s, beta0)
    betas = jnp.concatenate([beta1, beta2, beta3], axis=0).reshape(3, 1, -1)
    return _sweeps_q_fused(adj_q, s, cs, betas)
